# baseline (device time: 800064 ns/iter reference)
import jax
import jax.numpy as jnp
from jax import lax
from jax.experimental import pallas as pl
from jax.experimental.pallas import tpu as pltpu

N_DEV = 32
M = 8192
N_OUT = 4096
SEG = M // N_DEV
N_SUB = 4
N_RINGS = 2 * N_SUB
HQ = N_OUT // N_RINGS


def kernel(x, w_mat):
    def body(x_ref, w_ref, out_ref,
             rs_send, rs_recv, ag_send, ag_recv, stage,
             rs_ssem, rs_rsem, ag_ssem, ag_rsem,
             rs_credit, ag_credit, copy_sem):
        my = lax.axis_index("i")
        MESH = pl.DeviceIdType.MESH

        def l2c(k):
            z = k // 8
            j = lax.rem(k, 8)
            y = j // 2
            xi = lax.rem(j, 2)
            xx = jnp.where(lax.rem(y, 2) == 0, xi, 1 - xi)
            return xx, y, z

        def c2l(xx, y, z):
            return z * 8 + y * 2 + jnp.where(lax.rem(y, 2) == 0, xx, 1 - xx)

        def c2r(xx, y, z):
            i = 3 - y
            r0 = 4 * y + jnp.where(lax.rem(y, 2) == 0, z, 3 - z)
            r1 = 16 + 4 * i + jnp.where(lax.rem(i, 2) == 0, z, 3 - z)
            return jnp.where(xx == 0, r0, r1)

        def r2c(rr):
            xx = jnp.where(rr < 16, 0, 1)
            y0 = lax.rem(rr, 16) // 4
            q = lax.rem(rr, 4)
            y = jnp.where(xx == 0, y0, 3 - y0)
            z = jnp.where(lax.rem(y0, 2) == 0, q, 3 - q)
            return xx, y, z

        r = c2r(*l2c(my))
        rnext = c2l(*r2c(lax.rem(r + 1, N_DEV)))
        rprev = c2l(*r2c(lax.rem(r + N_DEV - 1, N_DEV)))

        def mod32(v):
            return lax.rem(v + 4 * N_DEV, N_DEV)

        RINGS = []
        for k in range(N_SUB):
            RINGS.append(dict(ri=k, col0=k * HQ, sgn=-1))
            RINGS.append(dict(ri=N_SUB + k, col0=(N_SUB + k) * HQ, sgn=+1))
        for cfg in RINGS:
            cfg["tgt"] = rnext if cfg["sgn"] < 0 else rprev
            cfg["ups"] = rprev if cfg["sgn"] < 0 else rnext

        barrier = pltpu.get_barrier_semaphore()
        pl.semaphore_signal(barrier, inc=1, device_id=(rprev,), device_id_type=MESH)
        pl.semaphore_signal(barrier, inc=1, device_id=(rnext,), device_id_type=MESH)
        pl.semaphore_wait(barrier, 2)

        for cfg in RINGS:
            pl.semaphore_signal(rs_credit.at[cfg["ri"]], inc=1,
                                device_id=(cfg["ups"],), device_id_type=MESH)
            pl.semaphore_signal(ag_credit.at[cfg["ri"]], inc=1,
                                device_id=(cfg["ups"],), device_id_type=MESH)

        def part(seg, col0):
            return jnp.dot(x_ref[pl.ds(seg * SEG, SEG), :],
                           w_ref[:, pl.ds(col0, HQ)],
                           preferred_element_type=jnp.float32)

        def rs_rdma(cfg):
            return pltpu.make_async_remote_copy(
                src_ref=rs_send.at[cfg["ri"]], dst_ref=rs_recv.at[cfg["ri"]],
                send_sem=rs_ssem.at[cfg["ri"]], recv_sem=rs_rsem.at[cfg["ri"]],
                device_id=(cfg["tgt"],), device_id_type=MESH)

        def ag_rdma(cfg):
            return pltpu.make_async_remote_copy(
                src_ref=ag_send.at[cfg["ri"]], dst_ref=ag_recv.at[cfg["ri"]],
                send_sem=ag_ssem.at[cfg["ri"]], recv_sem=ag_rsem.at[cfg["ri"]],
                device_id=(cfg["tgt"],), device_id_type=MESH)

        def out_copy(src_ref, seg, col0, sem):
            return pltpu.make_async_copy(
                src_ref,
                out_ref.at[pl.ds(seg * SEG, SEG), pl.ds(col0, HQ)],
                sem)

        for cfg in RINGS:
            rs_send[cfg["ri"]] = part(r, cfg["col0"]).astype(jnp.bfloat16)
            pl.semaphore_wait(rs_credit.at[cfg["ri"]], 1)
            rs_rdma(cfg).start()

        def rs_step(t, carry):
            parts = [part(mod32(r + cfg["sgn"] * (t + 1)), cfg["col0"])
                     for cfg in RINGS]
            for cfg, p in zip(RINGS, parts):
                ri = cfg["ri"]
                rdma = rs_rdma(cfg)
                rdma.wait_recv()
                new = rs_recv[ri][...] + p
                rdma.wait_send()
                rs_send[ri] = new.astype(jnp.bfloat16)
                pl.semaphore_signal(rs_credit.at[ri], inc=1,
                                    device_id=(cfg["ups"],), device_id_type=MESH)
                pl.semaphore_wait(rs_credit.at[ri], 1)
                rdma.start()
            return carry

        lax.fori_loop(0, N_DEV - 2, rs_step, 0)

        for cfg in RINGS:
            ri = cfg["ri"]
            rdma = rs_rdma(cfg)
            rdma.wait_recv()
            p = part(mod32(r + cfg["sgn"] * (N_DEV - 1)), cfg["col0"])
            fin = jnp.maximum(rs_recv[ri][...] + p, 0.0)
            rdma.wait_send()
            stage[ri] = fin
            ag_send[ri] = fin.astype(jnp.bfloat16)
            own = mod32(r - cfg["sgn"])
            out_copy(stage.at[ri], own, cfg["col0"], copy_sem.at[ri]).start()
            pl.semaphore_wait(ag_credit.at[ri], 1)
            ag_rdma(cfg).start()

        def ag_step(t, carry):
            for cfg in RINGS:
                ri = cfg["ri"]
                rdma = ag_rdma(cfg)
                rdma.wait_recv()
                s_prev = mod32(r + cfg["sgn"] * (t - 1))
                out_copy(stage.at[ri], s_prev, cfg["col0"], copy_sem.at[ri]).wait()
                rdma.wait_send()
                v = ag_recv[ri][...]
                ag_send[ri] = v
                stage[ri] = v.astype(jnp.float32)
                pl.semaphore_signal(ag_credit.at[ri], inc=1,
                                    device_id=(cfg["ups"],), device_id_type=MESH)
                pl.semaphore_wait(ag_credit.at[ri], 1)
                rdma.start()
                s = mod32(r + cfg["sgn"] * t)
                out_copy(stage.at[ri], s, cfg["col0"], copy_sem.at[ri]).start()
            return carry

        lax.fori_loop(0, N_DEV - 2, ag_step, 0)

        for cfg in RINGS:
            ri = cfg["ri"]
            t = N_DEV - 2
            rdma = ag_rdma(cfg)
            rdma.wait_recv()
            out_copy(stage.at[ri], mod32(r + cfg["sgn"] * (t - 1)), cfg["col0"],
                     copy_sem.at[ri]).wait()
            rdma.wait_send()
            stage[ri] = ag_recv[ri][...].astype(jnp.float32)
            s = mod32(r + cfg["sgn"] * t)
            out_copy(stage.at[ri], s, cfg["col0"], copy_sem.at[ri]).start()
        for cfg in RINGS:
            ri = cfg["ri"]
            s = mod32(r + cfg["sgn"] * (N_DEV - 2))
            out_copy(stage.at[ri], s, cfg["col0"], copy_sem.at[ri]).wait()

    bf16 = jnp.bfloat16
    f32 = jnp.float32
    return pl.pallas_call(
        body,
        out_shape=jax.ShapeDtypeStruct((M, N_OUT), f32),
        in_specs=[
            pl.BlockSpec(memory_space=pltpu.VMEM),
            pl.BlockSpec(memory_space=pltpu.VMEM),
        ],
        out_specs=pl.BlockSpec(memory_space=pl.ANY),
        scratch_shapes=[
            pltpu.VMEM((N_RINGS, SEG, HQ), bf16),
            pltpu.VMEM((N_RINGS, SEG, HQ), bf16),
            pltpu.VMEM((N_RINGS, SEG, HQ), bf16),
            pltpu.VMEM((N_RINGS, SEG, HQ), bf16),
            pltpu.VMEM((N_RINGS, SEG, HQ), f32),
            pltpu.SemaphoreType.DMA((N_RINGS,)),
            pltpu.SemaphoreType.DMA((N_RINGS,)),
            pltpu.SemaphoreType.DMA((N_RINGS,)),
            pltpu.SemaphoreType.DMA((N_RINGS,)),
            pltpu.SemaphoreType.REGULAR((N_RINGS,)),
            pltpu.SemaphoreType.REGULAR((N_RINGS,)),
            pltpu.SemaphoreType.DMA((N_RINGS,)),
        ],
        compiler_params=pltpu.CompilerParams(collective_id=0),
    )(x, w_mat)


# device time: 798805 ns/iter; 1.0016x vs baseline; 1.0016x over previous
import jax
import jax.numpy as jnp
from jax import lax
from jax.experimental import pallas as pl
from jax.experimental.pallas import tpu as pltpu

N_DEV = 32
M = 8192
N_OUT = 4096
SEG = M // N_DEV
HQ = N_OUT // 4


def kernel(x, w_mat):
    def body(x_ref, w_ref, out_ref,
             rs_send, rs_recv, ag_send, ag_recv, stage,
             rs_ssem, rs_rsem, ag_ssem, ag_rsem,
             rs_credit, ag_credit, copy_sem):
        my = lax.axis_index("i")
        MESH = pl.DeviceIdType.MESH

        def l2c(k):
            z = k // 8
            j = lax.rem(k, 8)
            y = j // 2
            xi = lax.rem(j, 2)
            xx = jnp.where(lax.rem(y, 2) == 0, xi, 1 - xi)
            return xx, y, z

        def c2l(xx, y, z):
            return z * 8 + y * 2 + jnp.where(lax.rem(y, 2) == 0, xx, 1 - xx)

        def c2r(xx, y, z):
            i = 3 - y
            r0 = 4 * y + jnp.where(lax.rem(y, 2) == 0, z, 3 - z)
            r1 = 16 + 4 * i + jnp.where(lax.rem(i, 2) == 0, z, 3 - z)
            return jnp.where(xx == 0, r0, r1)

        def r2c(rr):
            xx = jnp.where(rr < 16, 0, 1)
            y0 = lax.rem(rr, 16) // 4
            q = lax.rem(rr, 4)
            y = jnp.where(xx == 0, y0, 3 - y0)
            z = jnp.where(lax.rem(y0, 2) == 0, q, 3 - q)
            return xx, y, z

        r = c2r(*l2c(my))
        rnext = c2l(*r2c(lax.rem(r + 1, N_DEV)))
        rprev = c2l(*r2c(lax.rem(r + N_DEV - 1, N_DEV)))

        def mod32(v):
            return lax.rem(v + 4 * N_DEV, N_DEV)

        RINGS = [
            dict(ri=0, col0=0 * HQ, sgn=-1),
            dict(ri=2, col0=2 * HQ, sgn=+1),
            dict(ri=1, col0=1 * HQ, sgn=-1),
            dict(ri=3, col0=3 * HQ, sgn=+1),
        ]
        for cfg in RINGS:
            cfg["tgt"] = rnext if cfg["sgn"] < 0 else rprev
            cfg["ups"] = rprev if cfg["sgn"] < 0 else rnext

        barrier = pltpu.get_barrier_semaphore()
        pl.semaphore_signal(barrier, inc=1, device_id=(rprev,), device_id_type=MESH)
        pl.semaphore_signal(barrier, inc=1, device_id=(rnext,), device_id_type=MESH)
        pl.semaphore_wait(barrier, 2)

        for cfg in RINGS:
            pl.semaphore_signal(rs_credit.at[cfg["ri"]], inc=2,
                                device_id=(cfg["ups"],), device_id_type=MESH)
            pl.semaphore_signal(ag_credit.at[cfg["ri"]], inc=2,
                                device_id=(cfg["ups"],), device_id_type=MESH)

        def part(seg, col0):
            return jnp.dot(x_ref[pl.ds(seg * SEG, SEG), :],
                           w_ref[:, pl.ds(col0, HQ)],
                           preferred_element_type=jnp.float32)

        def rs_rdma(cfg, slot):
            return pltpu.make_async_remote_copy(
                src_ref=rs_send.at[cfg["ri"]],
                dst_ref=rs_recv.at[cfg["ri"], slot],
                send_sem=rs_ssem.at[cfg["ri"]],
                recv_sem=rs_rsem.at[cfg["ri"], slot],
                device_id=(cfg["tgt"],), device_id_type=MESH)

        def ag_rdma(cfg, slot):
            return pltpu.make_async_remote_copy(
                src_ref=ag_send.at[cfg["ri"]],
                dst_ref=ag_recv.at[cfg["ri"], slot],
                send_sem=ag_ssem.at[cfg["ri"]],
                recv_sem=ag_rsem.at[cfg["ri"], slot],
                device_id=(cfg["tgt"],), device_id_type=MESH)

        def out_copy(src_ref, seg, col0, sem):
            return pltpu.make_async_copy(
                src_ref,
                out_ref.at[pl.ds(seg * SEG, SEG), pl.ds(col0, HQ)],
                sem)

        for cfg in RINGS:
            rs_send[cfg["ri"]] = part(r, cfg["col0"]).astype(jnp.bfloat16)
            pl.semaphore_wait(rs_credit.at[cfg["ri"]], 1)
            rs_rdma(cfg, 0).start()

        def rs_hop(t, parity):
            parts = [part(mod32(r + cfg["sgn"] * (t + 1)), cfg["col0"])
                     for cfg in RINGS]
            for cfg, p in zip(RINGS, parts):
                ri = cfg["ri"]
                rdma = rs_rdma(cfg, parity)
                rdma.wait_recv()
                new = rs_recv[ri, parity][...] + p
                rdma.wait_send()
                rs_send[ri] = new.astype(jnp.bfloat16)

                @pl.when(t < N_DEV - 3)
                def _():
                    pl.semaphore_signal(rs_credit.at[ri], inc=1,
                                        device_id=(cfg["ups"],),
                                        device_id_type=MESH)
                pl.semaphore_wait(rs_credit.at[ri], 1)
                rs_rdma(cfg, 1 - parity).start()

        def rs_pair(u, carry):
            rs_hop(2 * u, 0)
            rs_hop(2 * u + 1, 1)
            return carry

        lax.fori_loop(0, (N_DEV - 2) // 2, rs_pair, 0)

        for cfg in RINGS:
            ri = cfg["ri"]
            rdma = rs_rdma(cfg, 0)
            rdma.wait_recv()
            p = part(mod32(r + cfg["sgn"] * (N_DEV - 1)), cfg["col0"])
            fin = jnp.maximum(rs_recv[ri, 0][...] + p, 0.0)
            rdma.wait_send()
            stage[ri] = fin
            ag_send[ri] = fin.astype(jnp.bfloat16)
            own = mod32(r - cfg["sgn"])
            out_copy(stage.at[ri], own, cfg["col0"], copy_sem.at[ri]).start()
            pl.semaphore_wait(ag_credit.at[ri], 1)
            ag_rdma(cfg, 0).start()

        def ag_hop(t, parity):
            for cfg in RINGS:
                ri = cfg["ri"]
                rdma = ag_rdma(cfg, parity)
                rdma.wait_recv()
                s_prev = mod32(r + cfg["sgn"] * (t - 1))
                out_copy(stage.at[ri], s_prev, cfg["col0"], copy_sem.at[ri]).wait()
                rdma.wait_send()
                v = ag_recv[ri, parity][...]
                ag_send[ri] = v
                stage[ri] = v.astype(jnp.float32)

                @pl.when(t < N_DEV - 3)
                def _():
                    pl.semaphore_signal(ag_credit.at[ri], inc=1,
                                        device_id=(cfg["ups"],),
                                        device_id_type=MESH)
                pl.semaphore_wait(ag_credit.at[ri], 1)
                ag_rdma(cfg, 1 - parity).start()
                s = mod32(r + cfg["sgn"] * t)
                out_copy(stage.at[ri], s, cfg["col0"], copy_sem.at[ri]).start()

        def ag_pair(u, carry):
            ag_hop(2 * u, 0)
            ag_hop(2 * u + 1, 1)
            return carry

        lax.fori_loop(0, (N_DEV - 2) // 2, ag_pair, 0)

        for cfg in RINGS:
            ri = cfg["ri"]
            t = N_DEV - 2
            rdma = ag_rdma(cfg, 0)
            rdma.wait_recv()
            out_copy(stage.at[ri], mod32(r + cfg["sgn"] * (t - 1)), cfg["col0"],
                     copy_sem.at[ri]).wait()
            rdma.wait_send()
            stage[ri] = ag_recv[ri, 0][...].astype(jnp.float32)
            s = mod32(r + cfg["sgn"] * t)
            out_copy(stage.at[ri], s, cfg["col0"], copy_sem.at[ri]).start()
        for cfg in RINGS:
            ri = cfg["ri"]
            s = mod32(r + cfg["sgn"] * (N_DEV - 2))
            out_copy(stage.at[ri], s, cfg["col0"], copy_sem.at[ri]).wait()

    bf16 = jnp.bfloat16
    f32 = jnp.float32
    return pl.pallas_call(
        body,
        out_shape=jax.ShapeDtypeStruct((M, N_OUT), f32),
        in_specs=[
            pl.BlockSpec(memory_space=pltpu.VMEM),
            pl.BlockSpec(memory_space=pltpu.VMEM),
        ],
        out_specs=pl.BlockSpec(memory_space=pl.ANY),
        scratch_shapes=[
            pltpu.VMEM((4, SEG, HQ), bf16),
            pltpu.VMEM((4, 2, SEG, HQ), bf16),
            pltpu.VMEM((4, SEG, HQ), bf16),
            pltpu.VMEM((4, 2, SEG, HQ), bf16),
            pltpu.VMEM((4, SEG, HQ), f32),
            pltpu.SemaphoreType.DMA((4,)),
            pltpu.SemaphoreType.DMA((4, 2)),
            pltpu.SemaphoreType.DMA((4,)),
            pltpu.SemaphoreType.DMA((4, 2)),
            pltpu.SemaphoreType.REGULAR((4,)),
            pltpu.SemaphoreType.REGULAR((4,)),
            pltpu.SemaphoreType.DMA((4,)),
        ],
        compiler_params=pltpu.CompilerParams(collective_id=0),
    )(x, w_mat)
